# baseline (speedup 1.0000x reference)
import functools

import jax
import jax.numpy as jnp
from jax import lax
from jax.experimental import pallas as pl
from jax.experimental.pallas import tpu as pltpu
from jax.experimental.pallas import tpu_sc as plsc

N = 10000
D_IN = 128
D = 256
NL = 6
NE = 4
E = 80000
CZ = D + D_IN

NPAD = 10240
ETOT = NE * E
CHUNK = 128
TILES = 16
NSC = 2
EPAD = 327680
ROWS2D = EPAD // CHUNK
RPT = ROWS2D // TILES
ST = 4
SROWS = RPT // ST
ART = NPAD // TILES


def _hw_body(h_ref, w_ref, o_ref):
    o_ref[...] = jnp.dot(h_ref[...], w_ref[0], preferred_element_type=jnp.float32)


_hw_call = pl.pallas_call(
    _hw_body,
    grid=(NSC, NE, 5),
    in_specs=[
        pl.BlockSpec((2000, D), lambda c, e, nb: (nb, 0)),
        pl.BlockSpec((1, D, 128), lambda c, e, nb: (e, 0, c)),
    ],
    out_specs=pl.BlockSpec((2000, 128), lambda c, e, nb: (c * 20 + e * 5 + nb, 0)),
    out_shape=jax.ShapeDtypeStruct((NSC * NE * N, 128), jnp.float32),
)


def _sc_agg(table, src3, dst2, zsrc, m_out, srcb, dstb, gbufa, gbufb, acc, sema, semb, ssema, ssemb):
    c = lax.axis_index("c")
    s = lax.axis_index("s")
    pltpu.sync_copy(zsrc, gbufa)
    for kk in range(ART // CHUNK):
        pltpu.sync_copy(gbufa, acc.at[pl.ds(s * ART + kk * CHUNK, CHUNK)])
    plsc.subcore_barrier()

    def stage_body(st, carry):
        pltpu.sync_copy(src3.at[pl.ds(c * ROWS2D + s * RPT + st * SROWS, SROWS)], srcb)
        pltpu.sync_copy(dst2.at[pl.ds(s * RPT + st * SROWS, SROWS)], dstb)
        pltpu.async_copy(table.at[srcb.at[0]], gbufa, sema)
        pltpu.async_copy(table.at[srcb.at[1]], gbufb, semb)

        def pair_body(p, carry2):
            k0 = 2 * p
            pltpu.make_async_copy(table.at[srcb.at[0]], gbufa, sema).wait()
            pltpu.async_copy(gbufa, acc.at[dstb.at[k0]], ssema, add=True)
            pltpu.make_async_copy(table.at[srcb.at[0]], gbufb, semb).wait()
            pltpu.async_copy(gbufb, acc.at[dstb.at[k0 + 1]], ssemb, add=True)
            ka = jnp.minimum(k0 + 2, SROWS - 1)
            kb = jnp.minimum(k0 + 3, SROWS - 1)
            pltpu.make_async_copy(gbufa, acc.at[dstb.at[0]], ssema).wait()
            pltpu.async_copy(table.at[srcb.at[ka]], gbufa, sema)
            pltpu.make_async_copy(gbufb, acc.at[dstb.at[0]], ssemb).wait()
            pltpu.async_copy(table.at[srcb.at[kb]], gbufb, semb)
            return carry2

        lax.fori_loop(0, SROWS // 2, pair_body, 0)
        pltpu.make_async_copy(table.at[srcb.at[0]], gbufa, sema).wait()
        pltpu.make_async_copy(table.at[srcb.at[0]], gbufb, semb).wait()
        return carry

    lax.fori_loop(0, ST, stage_body, 0)
    plsc.subcore_barrier()
    pltpu.sync_copy(
        acc.at[pl.ds(s * ART, ART)],
        m_out.at[pl.ds(s * ART, ART), pl.ds(c * 128, 128)],
    )


@functools.cache
def _sc_agg_call():
    mesh = plsc.VectorSubcoreMesh(core_axis_name="c", subcore_axis_name="s")
    return pl.kernel(
        _sc_agg,
        mesh=mesh,
        out_type=jax.ShapeDtypeStruct((NPAD, D), jnp.float32),
        scratch_types=[
            pltpu.VMEM((SROWS, CHUNK), jnp.int32),
            pltpu.VMEM((SROWS, CHUNK), jnp.int32),
            pltpu.VMEM((CHUNK, 128), jnp.float32),
            pltpu.VMEM((CHUNK, 128), jnp.float32),
            pltpu.VMEM_SHARED((NPAD, 128), jnp.float32),
            pltpu.SemaphoreType.DMA,
            pltpu.SemaphoreType.DMA,
            pltpu.SemaphoreType.DMA,
            pltpu.SemaphoreType.DMA,
        ],
    )


def _gru_body(m_ref, h_ref, wih_ref, whh_ref, bih_ref, bhh_ref, o_ref):
    m = m_ref[...]
    h = h_ref[...]
    cdims = (((1,), (1,)), ((), ()))
    gi = lax.dot_general(m, wih_ref[...], cdims, preferred_element_type=jnp.float32) + bih_ref[...]
    gh = lax.dot_general(h, whh_ref[...], cdims, preferred_element_type=jnp.float32) + bhh_ref[...]
    r = jax.nn.sigmoid(gi[:, 0:D] + gh[:, 0:D])
    z = jax.nn.sigmoid(gi[:, D:2 * D] + gh[:, D:2 * D])
    n = jnp.tanh(gi[:, 2 * D:3 * D] + r * gh[:, 2 * D:3 * D])
    o_ref[...] = (1.0 - z) * n + z * h


_gru_call = pl.pallas_call(
    _gru_body,
    grid=(10,),
    in_specs=[
        pl.BlockSpec((1000, D), lambda i: (i, 0)),
        pl.BlockSpec((1000, D), lambda i: (i, 0)),
        pl.BlockSpec((3 * D, D), lambda i: (0, 0)),
        pl.BlockSpec((3 * D, D), lambda i: (0, 0)),
        pl.BlockSpec((1, 3 * D), lambda i: (0, 0)),
        pl.BlockSpec((1, 3 * D), lambda i: (0, 0)),
    ],
    out_specs=pl.BlockSpec((1000, D), lambda i: (i, 0)),
    out_shape=jax.ShapeDtypeStruct((N, D), jnp.float32),
)


HB = 126
NHEAD = 20


def _conv_branch(xb, w1, b1, w2, b2):
    cdims = (((1,), (1,)), ((), ()))
    acc = lax.dot_general(xb[0:512], w1[0], cdims, preferred_element_type=jnp.float32)
    acc = acc + lax.dot_general(xb[1:513], w1[1], cdims, preferred_element_type=jnp.float32)
    acc = acc + lax.dot_general(xb[2:514], w1[2], cdims, preferred_element_type=jnp.float32)
    z1 = jnp.maximum(acc + b1, 0.0)
    z1m = z1.reshape(256, 2, z1.shape[1])
    ev = z1m[:, 0, :]
    ov = z1m[:, 1, :]
    z1p = jnp.maximum(jnp.maximum(ev[0:252], ov[0:252]), ev[1:253])
    z2 = lax.dot_general(z1p, w2[0], cdims, preferred_element_type=jnp.float32) + b2
    return jnp.max(z2.reshape(HB, 2, z2.shape[1]), axis=1)


def _head_body(hx_ref, wz1_ref, bz1_ref, wz2_ref, bz2_ref, wy1_ref, by1_ref,
               wy2_ref, by2_ref, mzw_ref, mzb_ref, myw_ref, myb_ref, o_ref):
    i0 = pl.program_id(0)
    cb = hx_ref[pl.ds(4 * HB * i0, 520), :]
    zf = _conv_branch(cb, wz1_ref[...], bz1_ref[...], wz2_ref[...], bz2_ref[...])
    yf = _conv_branch(cb[:, 0:D], wy1_ref[...], by1_ref[...], wy2_ref[...], by2_ref[...])
    zz = jnp.dot(zf, mzw_ref[...], preferred_element_type=jnp.float32) + mzb_ref[...]
    yy = jnp.dot(yf, myw_ref[...], preferred_element_type=jnp.float32) + myb_ref[...]
    prod = zz * yy
    gidx = HB * i0 + lax.broadcasted_iota(jnp.int32, (HB, 1), 0)
    prod = jnp.where(gidx < 2499, prod, 0.0)
    sv = jnp.sum(prod)
    svec = jnp.where(lax.broadcasted_iota(jnp.int32, (1, 128), 1) == 0, sv, 0.0)

    @pl.when(i0 == 0)
    def _():
        o_ref[...] = jnp.zeros((1, 128), jnp.float32)

    acc_o = o_ref[...] + svec
    o_ref[...] = acc_o

    @pl.when(i0 == NHEAD - 1)
    def _():
        o_ref[...] = jax.nn.sigmoid(acc_o / 2499.0)


_head_call = pl.pallas_call(
    _head_body,
    grid=(NHEAD,),
    in_specs=[
        pl.BlockSpec((10096, CZ), lambda i: (0, 0)),
        pl.BlockSpec((3, CZ, CZ), lambda i: (0, 0, 0)),
        pl.BlockSpec((1, CZ), lambda i: (0, 0)),
        pl.BlockSpec((1, CZ, CZ), lambda i: (0, 0, 0)),
        pl.BlockSpec((1, CZ), lambda i: (0, 0)),
        pl.BlockSpec((3, D, D), lambda i: (0, 0, 0)),
        pl.BlockSpec((1, D), lambda i: (0, 0)),
        pl.BlockSpec((1, D, D), lambda i: (0, 0, 0)),
        pl.BlockSpec((1, D), lambda i: (0, 0)),
        pl.BlockSpec((CZ, 1), lambda i: (0, 0)),
        pl.BlockSpec((1, 1), lambda i: (0, 0)),
        pl.BlockSpec((D, 1), lambda i: (0, 0)),
        pl.BlockSpec((1, 1), lambda i: (0, 0)),
    ],
    out_specs=pl.BlockSpec((1, 128), lambda i: (0, 0)),
    out_shape=jax.ShapeDtypeStruct((1, 128), jnp.float32),
)


def kernel(x, ast_edge_index, cfg_edge_index, ddg_edge_index, ncs_edge_index,
           W_ggnn, w_ih, w_hh, b_ih, b_hh,
           conv_z1_w, conv_z1_b, conv_z2_w, conv_z2_b,
           conv_y1_w, conv_y1_b, conv_y2_w, conv_y2_b,
           mlp_z_w, mlp_z_b, mlp_y_w, mlp_y_b):
    edges = [ast_edge_index, cfg_edge_index, ddg_edge_index, ncs_edge_index]
    src = jnp.concatenate([ei[0] + e * N for e, ei in enumerate(edges)])
    dst = jnp.concatenate([ei[1] for ei in edges])
    pad_i = jnp.arange(EPAD - ETOT, dtype=jnp.int32)
    src = jnp.concatenate([src, pad_i % N])
    dst = jnp.concatenate([dst, N + pad_i % (NPAD - N)])
    src3 = jnp.concatenate([src, src + NE * N]).reshape(2 * ROWS2D, CHUNK)
    dst2 = dst.reshape(ROWS2D, CHUNK)
    zsrc = jnp.zeros((CHUNK, 128), jnp.float32)

    h = jnp.pad(x, ((0, 0), (0, D - D_IN)))
    bih2 = b_ih.reshape(1, 3 * D)
    bhh2 = b_hh.reshape(1, 3 * D)
    for t in range(NL):
        hw = _hw_call(h, W_ggnn[t])
        m = _sc_agg_call()(hw, src3, dst2, zsrc)
        h = _gru_call(m[:N], h, w_ih, w_hh, bih2, bhh2)

    hx = jnp.pad(jnp.concatenate([h, x], axis=1), ((0, 96), (0, 0)))
    out = _head_call(
        hx,
        jnp.transpose(conv_z1_w, (2, 0, 1)), conv_z1_b.reshape(1, CZ),
        jnp.transpose(conv_z2_w, (2, 0, 1)), conv_z2_b.reshape(1, CZ),
        jnp.transpose(conv_y1_w, (2, 0, 1)), conv_y1_b.reshape(1, D),
        jnp.transpose(conv_y2_w, (2, 0, 1)), conv_y2_b.reshape(1, D),
        mlp_z_w, mlp_z_b.reshape(1, 1),
        mlp_y_w, mlp_y_b.reshape(1, 1),
    )
    return out[0, 0]


# speedup vs baseline: 1.2617x; 1.2617x over previous
import functools

import jax
import jax.numpy as jnp
from jax import lax
from jax.experimental import pallas as pl
from jax.experimental.pallas import tpu as pltpu
from jax.experimental.pallas import tpu_sc as plsc

N = 10000
D_IN = 128
D = 256
NL = 6
NE = 4
E = 80000
CZ = D + D_IN

NPAD = 10240
ETOT = NE * E
CHUNK = 128
TILES = 16
NSC = 2
EPAD = 327680
ROWS2D = EPAD // CHUNK
RPT = ROWS2D // TILES
ST = 4
SROWS = RPT // ST
ART = NPAD // TILES


def _hw_body(h_ref, w_ref, o_ref):
    o_ref[...] = jnp.dot(h_ref[...], w_ref[0], preferred_element_type=jnp.float32)


_hw_call = pl.pallas_call(
    _hw_body,
    grid=(NSC, NE, 5),
    in_specs=[
        pl.BlockSpec((2000, D), lambda c, e, nb: (nb, 0)),
        pl.BlockSpec((1, D, 128), lambda c, e, nb: (e, 0, c)),
    ],
    out_specs=pl.BlockSpec((2000, 128), lambda c, e, nb: (c * 20 + e * 5 + nb, 0)),
    out_shape=jax.ShapeDtypeStruct((NSC * NE * N, 128), jnp.float32),
)


def _sc_agg(table, src3, dst2, zsrc, m_out, srcb, dstb, gbufa, gbufb, acc, sema, semb):
    c = lax.axis_index("c")
    s = lax.axis_index("s")
    pltpu.sync_copy(zsrc, gbufa)
    for kk in range(ART // CHUNK):
        pltpu.sync_copy(gbufa, acc.at[pl.ds(s * ART + kk * CHUNK, CHUNK)])
    plsc.subcore_barrier()

    def stage_body(st, carry):
        pltpu.sync_copy(src3.at[pl.ds(c * ROWS2D + s * RPT + st * SROWS, SROWS)], srcb)
        pltpu.sync_copy(dst2.at[pl.ds(s * RPT + st * SROWS, SROWS)], dstb)
        pltpu.async_copy(table.at[srcb.at[0]], gbufa, sema)

        def pair_body(p, carry2):
            k0 = 2 * p
            pltpu.async_copy(table.at[srcb.at[k0 + 1]], gbufb, semb)
            pltpu.make_async_copy(table.at[srcb.at[0]], gbufa, sema).wait()
            pltpu.sync_copy(gbufa, acc.at[dstb.at[k0]], add=True)
            knext = jnp.minimum(k0 + 2, SROWS - 1)
            pltpu.async_copy(table.at[srcb.at[knext]], gbufa, sema)
            pltpu.make_async_copy(table.at[srcb.at[0]], gbufb, semb).wait()
            pltpu.sync_copy(gbufb, acc.at[dstb.at[k0 + 1]], add=True)
            return carry2

        lax.fori_loop(0, SROWS // 2, pair_body, 0)
        pltpu.make_async_copy(table.at[srcb.at[0]], gbufa, sema).wait()
        return carry

    lax.fori_loop(0, ST, stage_body, 0)
    plsc.subcore_barrier()
    pltpu.sync_copy(
        acc.at[pl.ds(s * ART, ART)],
        m_out.at[pl.ds(s * ART, ART), pl.ds(c * 128, 128)],
    )


@functools.cache
def _sc_agg_call():
    mesh = plsc.VectorSubcoreMesh(core_axis_name="c", subcore_axis_name="s")
    return pl.kernel(
        _sc_agg,
        mesh=mesh,
        out_type=jax.ShapeDtypeStruct((NPAD, D), jnp.float32),
        scratch_types=[
            pltpu.VMEM((SROWS, CHUNK), jnp.int32),
            pltpu.VMEM((SROWS, CHUNK), jnp.int32),
            pltpu.VMEM((CHUNK, 128), jnp.float32),
            pltpu.VMEM((CHUNK, 128), jnp.float32),
            pltpu.VMEM_SHARED((NPAD, 128), jnp.float32),
            pltpu.SemaphoreType.DMA,
            pltpu.SemaphoreType.DMA,
        ],
    )


def _gru_body(m_ref, h_ref, wih_ref, whh_ref, bih_ref, bhh_ref, o_ref):
    m = m_ref[...]
    h = h_ref[...]
    cdims = (((1,), (1,)), ((), ()))
    gi = lax.dot_general(m, wih_ref[...], cdims, preferred_element_type=jnp.float32) + bih_ref[...]
    gh = lax.dot_general(h, whh_ref[...], cdims, preferred_element_type=jnp.float32) + bhh_ref[...]
    r = jax.nn.sigmoid(gi[:, 0:D] + gh[:, 0:D])
    z = jax.nn.sigmoid(gi[:, D:2 * D] + gh[:, D:2 * D])
    n = jnp.tanh(gi[:, 2 * D:3 * D] + r * gh[:, 2 * D:3 * D])
    o_ref[...] = (1.0 - z) * n + z * h


_gru_call = pl.pallas_call(
    _gru_body,
    grid=(10,),
    in_specs=[
        pl.BlockSpec((1000, D), lambda i: (i, 0)),
        pl.BlockSpec((1000, D), lambda i: (i, 0)),
        pl.BlockSpec((3 * D, D), lambda i: (0, 0)),
        pl.BlockSpec((3 * D, D), lambda i: (0, 0)),
        pl.BlockSpec((1, 3 * D), lambda i: (0, 0)),
        pl.BlockSpec((1, 3 * D), lambda i: (0, 0)),
    ],
    out_specs=pl.BlockSpec((1000, D), lambda i: (i, 0)),
    out_shape=jax.ShapeDtypeStruct((N, D), jnp.float32),
)


HB = 126
NHEAD = 20


def _conv_branch(xb, w1, b1, w2, b2):
    cdims = (((1,), (1,)), ((), ()))
    acc = lax.dot_general(xb[0:512], w1[0], cdims, preferred_element_type=jnp.float32)
    acc = acc + lax.dot_general(xb[1:513], w1[1], cdims, preferred_element_type=jnp.float32)
    acc = acc + lax.dot_general(xb[2:514], w1[2], cdims, preferred_element_type=jnp.float32)
    z1 = jnp.maximum(acc + b1, 0.0)
    z1m = z1.reshape(256, 2, z1.shape[1])
    ev = z1m[:, 0, :]
    ov = z1m[:, 1, :]
    z1p = jnp.maximum(jnp.maximum(ev[0:252], ov[0:252]), ev[1:253])
    z2 = lax.dot_general(z1p, w2[0], cdims, preferred_element_type=jnp.float32) + b2
    return jnp.max(z2.reshape(HB, 2, z2.shape[1]), axis=1)


def _head_body(hx_ref, wz1_ref, bz1_ref, wz2_ref, bz2_ref, wy1_ref, by1_ref,
               wy2_ref, by2_ref, mzw_ref, mzb_ref, myw_ref, myb_ref, o_ref):
    i0 = pl.program_id(0)
    cb = hx_ref[pl.ds(4 * HB * i0, 520), :]
    zf = _conv_branch(cb, wz1_ref[...], bz1_ref[...], wz2_ref[...], bz2_ref[...])
    yf = _conv_branch(cb[:, 0:D], wy1_ref[...], by1_ref[...], wy2_ref[...], by2_ref[...])
    zz = jnp.dot(zf, mzw_ref[...], preferred_element_type=jnp.float32) + mzb_ref[...]
    yy = jnp.dot(yf, myw_ref[...], preferred_element_type=jnp.float32) + myb_ref[...]
    prod = zz * yy
    gidx = HB * i0 + lax.broadcasted_iota(jnp.int32, (HB, 1), 0)
    prod = jnp.where(gidx < 2499, prod, 0.0)
    sv = jnp.sum(prod)
    svec = jnp.where(lax.broadcasted_iota(jnp.int32, (1, 128), 1) == 0, sv, 0.0)

    @pl.when(i0 == 0)
    def _():
        o_ref[...] = jnp.zeros((1, 128), jnp.float32)

    acc_o = o_ref[...] + svec
    o_ref[...] = acc_o

    @pl.when(i0 == NHEAD - 1)
    def _():
        o_ref[...] = jax.nn.sigmoid(acc_o / 2499.0)


_head_call = pl.pallas_call(
    _head_body,
    grid=(NHEAD,),
    in_specs=[
        pl.BlockSpec((10096, CZ), lambda i: (0, 0)),
        pl.BlockSpec((3, CZ, CZ), lambda i: (0, 0, 0)),
        pl.BlockSpec((1, CZ), lambda i: (0, 0)),
        pl.BlockSpec((1, CZ, CZ), lambda i: (0, 0, 0)),
        pl.BlockSpec((1, CZ), lambda i: (0, 0)),
        pl.BlockSpec((3, D, D), lambda i: (0, 0, 0)),
        pl.BlockSpec((1, D), lambda i: (0, 0)),
        pl.BlockSpec((1, D, D), lambda i: (0, 0, 0)),
        pl.BlockSpec((1, D), lambda i: (0, 0)),
        pl.BlockSpec((CZ, 1), lambda i: (0, 0)),
        pl.BlockSpec((1, 1), lambda i: (0, 0)),
        pl.BlockSpec((D, 1), lambda i: (0, 0)),
        pl.BlockSpec((1, 1), lambda i: (0, 0)),
    ],
    out_specs=pl.BlockSpec((1, 128), lambda i: (0, 0)),
    out_shape=jax.ShapeDtypeStruct((1, 128), jnp.float32),
)


def kernel(x, ast_edge_index, cfg_edge_index, ddg_edge_index, ncs_edge_index,
           W_ggnn, w_ih, w_hh, b_ih, b_hh,
           conv_z1_w, conv_z1_b, conv_z2_w, conv_z2_b,
           conv_y1_w, conv_y1_b, conv_y2_w, conv_y2_b,
           mlp_z_w, mlp_z_b, mlp_y_w, mlp_y_b):
    edges = [ast_edge_index, cfg_edge_index, ddg_edge_index, ncs_edge_index]
    src = jnp.concatenate([ei[0] + e * N for e, ei in enumerate(edges)])
    dst = jnp.concatenate([ei[1] for ei in edges])
    pad_i = jnp.arange(EPAD - ETOT, dtype=jnp.int32)
    src = jnp.concatenate([src, pad_i % N])
    dst = jnp.concatenate([dst, N + pad_i % (NPAD - N)])
    src3 = jnp.concatenate([src, src + NE * N]).reshape(2 * ROWS2D, CHUNK)
    dst2 = dst.reshape(ROWS2D, CHUNK)
    zsrc = jnp.zeros((CHUNK, 128), jnp.float32)

    h = jnp.pad(x, ((0, 0), (0, D - D_IN)))
    bih2 = b_ih.reshape(1, 3 * D)
    bhh2 = b_hh.reshape(1, 3 * D)
    for t in range(NL):
        hw = _hw_call(h, W_ggnn[t])
        m = _sc_agg_call()(hw, src3, dst2, zsrc)
        h = _gru_call(m[:N], h, w_ih, w_hh, bih2, bhh2)

    hx = jnp.pad(jnp.concatenate([h, x], axis=1), ((0, 96), (0, 0)))
    out = _head_call(
        hx,
        jnp.transpose(conv_z1_w, (2, 0, 1)), conv_z1_b.reshape(1, CZ),
        jnp.transpose(conv_z2_w, (2, 0, 1)), conv_z2_b.reshape(1, CZ),
        jnp.transpose(conv_y1_w, (2, 0, 1)), conv_y1_b.reshape(1, D),
        jnp.transpose(conv_y2_w, (2, 0, 1)), conv_y2_b.reshape(1, D),
        mlp_z_w, mlp_z_b.reshape(1, 1),
        mlp_y_w, mlp_y_b.reshape(1, 1),
    )
    return out[0, 0]
